# baseline (speedup 1.0000x reference)
import functools

import jax
import jax.numpy as jnp
from jax import lax
from jax.experimental import pallas as pl
from jax.experimental.pallas import tpu as pltpu
from jax.experimental.pallas import tpu_sc as plsc

N_FIELDS = 26
VOCAB = 100
DIM = 128
BATCH = 16384

NC = 2
NS = 16
NW = NC * NS
L = 16

ELEMS_PER_W = BATCH // NW
NE = 8
ITERS = ELEMS_PER_W // NE
CROWS = NE * N_FIELDS
OUTW = DIM * N_FIELDS


def _sc_embed(xflat, U):
  mesh = plsc.VectorSubcoreMesh(core_axis_name="c", subcore_axis_name="s")

  @functools.partial(
      pl.kernel,
      mesh=mesh,
      out_type=jax.ShapeDtypeStruct((BATCH * OUTW,), jnp.float32),
      scratch_types=[
          pltpu.VMEM((CROWS,), jnp.int32),
          pltpu.VMEM((CROWS, DIM), jnp.float32),
          pltpu.VMEM((NE * OUTW,), jnp.float32),
          pltpu.SemaphoreType.DMA,
      ],
      compiler_params=pltpu.CompilerParams(needs_layout_passes=False),
  )
  def k(x_hbm, u_hbm, o_hbm, idx_v, rows_v, out_v, sem):
    wid = lax.axis_index("s") * NC + lax.axis_index("c")
    ebase0 = wid * ELEMS_PER_W
    lane = lax.iota(jnp.int32, L)
    lane26 = lane * N_FIELDS

    def chunk(t, carry):
      ebase = ebase0 + t * NE
      pltpu.sync_copy(x_hbm.at[pl.ds(ebase * N_FIELDS, CROWS)], idx_v)
      for j in range(CROWS // L):
        fld = lax.rem(j * L + lane, N_FIELDS) * VOCAB
        sl = pl.ds(j * L, L)
        idx_v[sl] = idx_v[sl] + fld
      pltpu.async_copy(u_hbm.at[idx_v], rows_v, sem).wait()

      def elem(e, c2):
        rowbase = e * N_FIELDS
        outbase = e * OUTW

        def field(c, c3):
          row = rowbase + c
          ob = outbase + c
          for db in range(DIM // L):
            v = rows_v[row, pl.ds(db * L, L)]
            plsc.store_scatter(out_v, [lane26 + (ob + db * L * N_FIELDS)], v)
          return c3

        return lax.fori_loop(0, N_FIELDS, field, c2)

      lax.fori_loop(0, NE, elem, 0)
      pltpu.sync_copy(out_v, o_hbm.at[pl.ds(ebase * OUTW, NE * OUTW)])
      return carry

    lax.fori_loop(0, ITERS, chunk, 0)

  return k(xflat, U)


def kernel(x, W):
  xflat = x.reshape(-1).astype(jnp.int32)
  U = W.reshape(N_FIELDS * VOCAB, DIM)
  out = _sc_embed(xflat, U)
  return out.reshape(BATCH, DIM, N_FIELDS)


# speedup vs baseline: 1.1652x; 1.1652x over previous
import functools

import jax
import jax.numpy as jnp
from jax import lax
from jax.experimental import pallas as pl
from jax.experimental.pallas import tpu as pltpu
from jax.experimental.pallas import tpu_sc as plsc

N_FIELDS = 26
VOCAB = 100
DIM = 128
BATCH = 16384

NC = 2
NS = 16
NW = NC * NS
L = 16

ELEMS_PER_W = BATCH // NW
NE = 8
ITERS = ELEMS_PER_W // NE
CROWS = NE * N_FIELDS
OUTW = DIM * N_FIELDS


def _sc_embed(xflat, U):
  mesh = plsc.VectorSubcoreMesh(core_axis_name="c", subcore_axis_name="s")

  @functools.partial(
      pl.kernel,
      mesh=mesh,
      out_type=jax.ShapeDtypeStruct((BATCH * OUTW,), jnp.float32),
      scratch_types=[
          pltpu.VMEM((CROWS,), jnp.int32),
          pltpu.VMEM((CROWS,), jnp.int32),
          pltpu.VMEM((CROWS, DIM), jnp.float32),
          pltpu.VMEM((CROWS, DIM), jnp.float32),
          pltpu.VMEM((NE * OUTW,), jnp.float32),
          pltpu.VMEM((NE * OUTW,), jnp.float32),
          pltpu.SemaphoreType.DMA,
          pltpu.SemaphoreType.DMA,
          pltpu.SemaphoreType.DMA,
          pltpu.SemaphoreType.DMA,
      ],
      compiler_params=pltpu.CompilerParams(needs_layout_passes=False),
  )
  def k(x_hbm, u_hbm, o_hbm, idxA, idxB, rowsA, rowsB, outA, outB,
        g0, g1, o0, o1):
    idx_ = (idxA, idxB)
    rows_ = (rowsA, rowsB)
    out_ = (outA, outB)
    gsem = (g0, g1)
    osem = (o0, o1)
    wid = lax.axis_index("s") * NC + lax.axis_index("c")
    ebase0 = wid * ELEMS_PER_W
    lane = lax.iota(jnp.int32, L)
    lane26 = lane * N_FIELDS

    def prefetch(t, b):
      ebase = ebase0 + t * NE
      pltpu.sync_copy(x_hbm.at[pl.ds(ebase * N_FIELDS, CROWS)], idx_[b])
      for j in range(CROWS // L):
        fld = lax.rem(j * L + lane, N_FIELDS) * VOCAB
        sl = pl.ds(j * L, L)
        idx_[b][sl] = idx_[b][sl] + fld
      pltpu.make_async_copy(u_hbm.at[idx_[b]], rows_[b], gsem[b]).start()

    def wait_gather(b):
      pltpu.make_async_copy(u_hbm.at[idx_[b]], rows_[b], gsem[b]).wait()

    def out_copy(t, b):
      ebase = ebase0 + t * NE
      return pltpu.make_async_copy(
          out_[b], o_hbm.at[pl.ds(ebase * OUTW, NE * OUTW)], osem[b])

    def transpose(b):
      rows_b = rows_[b]
      out_b = out_[b]

      @plsc.parallel_loop(0, NE)
      def _(e):
        rowb = e * N_FIELDS
        ob = e * OUTW
        for c in range(N_FIELDS):
          row = rowb + c
          for db in range(DIM // L):
            v = rows_b[row, pl.ds(db * L, L)]
            plsc.store_scatter(
                out_b, [lane26 + (ob + db * L * N_FIELDS + c)], v)

    prefetch(0, 0)
    prefetch(1, 1)

    def body(tt, carry):
      for b in range(2):
        t = tt * 2 + b
        wait_gather(b)

        @pl.when(tt > 0)
        def _():
          out_copy(t - 2, b).wait()

        transpose(b)
        out_copy(t, b).start()

        @pl.when(t + 2 < ITERS)
        def _():
          prefetch(t + 2, b)

      return carry

    lax.fori_loop(0, ITERS // 2, body, 0)
    out_copy(ITERS - 2, 0).wait()
    out_copy(ITERS - 1, 1).wait()

  return k(xflat, U)


def kernel(x, W):
  xflat = x.reshape(-1).astype(jnp.int32)
  U = W.reshape(N_FIELDS * VOCAB, DIM)
  out = _sc_embed(xflat, U)
  return out.reshape(BATCH, DIM, N_FIELDS)
